# baseline (speedup 1.0000x reference)
import functools

import jax
import jax.numpy as jnp
from jax import lax
from jax.experimental import pallas as pl
from jax.experimental.pallas import tpu as pltpu
from jax.experimental.pallas import tpu_sc as plsc

_NC = 2
_NS = 16
_L = 16
_NW = _NC * _NS

_N = 4096
_E = 262144
_EW = _E // _NW
_CH = 128
_NCH = _EW // _CH
_VPC = _CH // _L
_QD = 8
_NG = 4
_LAMBDA = 2.0
_ZCH = _N // _NS


def _scatter_chunk(probv, hist_sh, scs, ssem, j):
    return pltpu.make_async_copy(probv.at[j], hist_sh.at[scs.at[j]], ssem)


def _sc_body(dist_hbm, eidx_hbm, logit_hbm, hist_out, pd_out,
             sdv, logv, scs, idxv, gathv, probv, pdv, zv, hist_sh,
             sem, ssem, *gsems):
    c = lax.axis_index("c")
    s = lax.axis_index("s")
    wid = s * _NC + c
    base = wid * _EW

    cp1 = pltpu.async_copy(eidx_hbm.at[pl.ds(2 * base, 2 * _EW)], sdv, sem)
    cp3 = pltpu.async_copy(logit_hbm.at[pl.ds(base, _EW)], logv, sem)

    for i in range(_ZCH // _L):
        zv[pl.ds(i * _L, _L)] = jnp.zeros((_L,), jnp.float32)
    cp1.wait()
    cp3.wait()
    pltpu.sync_copy(zv, hist_sh.at[pl.ds(s * _ZCH, _ZCH)])
    plsc.subcore_barrier()

    def _ibody(j, _):
        for b in range(_VPC):
            o = b * _L
            sv = sdv[pl.ds(2 * j * _CH + o, _L)]
            dv = sdv[pl.ds((2 * j + 1) * _CH + o, _L)]
            phys = ((sv >> 3) << 15) + ((dv >> 7) << 10) + ((sv & 7) << 7) \
                + (dv & 127)
            idxv[pl.ds(j * _CH + o, _L)] = phys
            scs[j, pl.ds(o, _L)] = sv
        return 0

    _CPG = _NCH // _NG
    _EG = _EW // _NG
    for g in range(_NG):
        lax.fori_loop(g * _CPG, (g + 1) * _CPG, _ibody, 0)
        pltpu.async_copy(
            dist_hbm.at[idxv.at[pl.ds(g * _EG, _EG)]],
            gathv.at[pl.ds(g * _EG, _EG)], gsems[g])

    def _chunk(j, _):
        for b in range(_VPC):
            x = logv[pl.ds(j * _CH + b * _L, _L)]
            probv[j, pl.ds(b * _L, _L)] = 1.0 / (1.0 + jnp.exp(-x))
        pltpu.async_copy(probv.at[j], hist_sh.at[scs.at[j]], ssem, add=True)

        @pl.when(j >= _QD)
        def _():
            _scatter_chunk(probv, hist_sh, scs, ssem, j - _QD).wait()
        return 0
    lax.fori_loop(0, _NCH, _chunk, 0)
    for k in range(_QD):
        _scatter_chunk(probv, hist_sh, scs, ssem, _NCH - _QD + k).wait()

    def _dbody(j, acc):
        for b in range(_VPC):
            acc = acc + probv[j, pl.ds(b * _L, _L)] \
                * gathv[pl.ds(j * _CH + b * _L, _L)]
        return acc
    acc = jnp.zeros((_L,), jnp.float32)
    for g in range(_NG):
        pltpu.make_async_copy(
            dist_hbm.at[idxv.at[pl.ds(g * _EG, _EG)]],
            gathv.at[pl.ds(g * _EG, _EG)], gsems[g]).wait()
        acc = lax.fori_loop(g * _CPG, (g + 1) * _CPG, _dbody, acc)
    pdv[...] = acc
    pltpu.sync_copy(pdv, pd_out.at[pl.ds(wid * _L, _L)])

    plsc.subcore_barrier()

    @pl.when(s == 0)
    def _():
        pltpu.sync_copy(hist_sh, hist_out.at[pl.ds(c * _N, _N)])


_sc_call = functools.partial(
    pl.kernel,
    out_type=[
        jax.ShapeDtypeStruct((_NC * _N,), jnp.float32),
        jax.ShapeDtypeStruct((_NW * _L,), jnp.float32),
    ],
    mesh=plsc.VectorSubcoreMesh(
        core_axis_name="c", subcore_axis_name="s",
        num_cores=_NC, num_subcores=_NS),
    scratch_types=[
        pltpu.VMEM((2 * _EW,), jnp.int32),
        pltpu.VMEM((_EW,), jnp.float32),
        pltpu.VMEM((_NCH, _CH), jnp.int32),
        pltpu.VMEM((_EW,), jnp.int32),
        pltpu.VMEM((_EW,), jnp.float32),
        pltpu.VMEM((_NCH, _CH), jnp.float32),
        pltpu.VMEM((_L,), jnp.float32),
        pltpu.VMEM((_ZCH,), jnp.float32),
        pltpu.VMEM_SHARED((_N,), jnp.float32),
        pltpu.SemaphoreType.DMA,
        pltpu.SemaphoreType.DMA,
        pltpu.SemaphoreType.DMA,
        pltpu.SemaphoreType.DMA,
        pltpu.SemaphoreType.DMA,
        pltpu.SemaphoreType.DMA,
    ],
)(_sc_body)


def _tc_reduce(h_ref, pd_ref, out_ref):
    wd = h_ref[0:_N] + h_ref[_N:2 * _N]
    d = wd - 2.0
    loss_deg = jnp.sum(d * d) * (1.0 / _N)
    loss_dist = jnp.sum(pd_ref[...])
    out_ref[0, 0] = loss_dist + _LAMBDA * loss_deg


_tc_call = pl.pallas_call(
    _tc_reduce,
    out_shape=jax.ShapeDtypeStruct((1, 1), jnp.float32),
    out_specs=pl.BlockSpec(memory_space=pltpu.SMEM),
)


def kernel(edge_logits, distances, edge_index, num_nodes):
    del num_nodes
    dist_flat = (distances.reshape(_N // 8, 8, _N // 128, 128)
                 .transpose(0, 2, 1, 3).reshape(_N * _N))
    eidx_flat = (edge_index.astype(jnp.int32).reshape(2, _E // 128, 128)
                 .transpose(1, 0, 2).reshape(2 * _E))
    hist, pd = _sc_call(dist_flat, eidx_flat, edge_logits)
    res = _tc_call(hist, pd)
    return res[0, 0]


# speedup vs baseline: 1.0491x; 1.0491x over previous
import functools

import jax
import jax.numpy as jnp
from jax import lax
from jax.experimental import pallas as pl
from jax.experimental.pallas import tpu as pltpu
from jax.experimental.pallas import tpu_sc as plsc

_NC = 2
_NS = 16
_L = 16
_NW = _NC * _NS

_N = 4096
_E = 262144
_EW = _E // _NW
_CH = 128
_NCH = _EW // _CH
_VPC = _CH // _L
_QD = 8
_NG = 2
_LAMBDA = 2.0
_ZCH = _N // _NS


def _scatter_chunk(probv, hist_sh, scs, ssem, j):
    return pltpu.make_async_copy(probv.at[j], hist_sh.at[scs.at[j]], ssem)


def _sc_body(dist_hbm, src_hbm, dst_hbm, logit_hbm, hist_out, pd_out,
             srcv, dstv, logv, scs, idxv, gathv, probv, pdv, zv, hist_sh,
             sem, ssem, *gsems):
    c = lax.axis_index("c")
    s = lax.axis_index("s")
    wid = s * _NC + c
    base = wid * _EW

    cp1 = pltpu.async_copy(src_hbm.at[pl.ds(base, _EW)], srcv, sem)
    cp2 = pltpu.async_copy(dst_hbm.at[pl.ds(base, _EW)], dstv, sem)
    cp3 = pltpu.async_copy(logit_hbm.at[pl.ds(base, _EW)], logv, sem)

    for i in range(_ZCH // _L):
        zv[pl.ds(i * _L, _L)] = jnp.zeros((_L,), jnp.float32)
    cp1.wait()
    cp2.wait()
    cp3.wait()
    pltpu.sync_copy(zv, hist_sh.at[pl.ds(s * _ZCH, _ZCH)])
    plsc.subcore_barrier()

    def _ibody(j, _):
        for b in range(_VPC):
            o = b * _L
            off = j * _CH + o
            sv = srcv[pl.ds(off, _L)]
            dv = dstv[pl.ds(off, _L)]
            phys = ((sv >> 3) << 15) + ((dv >> 7) << 10) + ((sv & 7) << 7) \
                + (dv & 127)
            idxv[pl.ds(off, _L)] = phys
            scs[j, pl.ds(o, _L)] = sv
        return 0

    _CPG = _NCH // _NG
    _EG = _EW // _NG
    for g in range(_NG):
        lax.fori_loop(g * _CPG, (g + 1) * _CPG, _ibody, 0)
        pltpu.async_copy(
            dist_hbm.at[idxv.at[pl.ds(g * _EG, _EG)]],
            gathv.at[pl.ds(g * _EG, _EG)], gsems[g])

    def _chunk(j, _):
        for b in range(_VPC):
            x = logv[pl.ds(j * _CH + b * _L, _L)]
            probv[j, pl.ds(b * _L, _L)] = 1.0 / (1.0 + jnp.exp(-x))
        pltpu.async_copy(probv.at[j], hist_sh.at[scs.at[j]], ssem, add=True)

        @pl.when(j >= _QD)
        def _():
            _scatter_chunk(probv, hist_sh, scs, ssem, j - _QD).wait()
        return 0
    lax.fori_loop(0, _NCH, _chunk, 0)
    for k in range(_QD):
        _scatter_chunk(probv, hist_sh, scs, ssem, _NCH - _QD + k).wait()

    def _dbody(j, acc):
        for b in range(_VPC):
            acc = acc + probv[j, pl.ds(b * _L, _L)] \
                * gathv[pl.ds(j * _CH + b * _L, _L)]
        return acc
    acc = jnp.zeros((_L,), jnp.float32)
    for g in range(_NG):
        pltpu.make_async_copy(
            dist_hbm.at[idxv.at[pl.ds(g * _EG, _EG)]],
            gathv.at[pl.ds(g * _EG, _EG)], gsems[g]).wait()
        acc = lax.fori_loop(g * _CPG, (g + 1) * _CPG, _dbody, acc)
    pdv[...] = acc
    pltpu.sync_copy(pdv, pd_out.at[pl.ds(wid * _L, _L)])

    plsc.subcore_barrier()

    @pl.when(s == 0)
    def _():
        pltpu.sync_copy(hist_sh, hist_out.at[pl.ds(c * _N, _N)])


_sc_call = functools.partial(
    pl.kernel,
    out_type=[
        jax.ShapeDtypeStruct((_NC * _N,), jnp.float32),
        jax.ShapeDtypeStruct((_NW * _L,), jnp.float32),
    ],
    mesh=plsc.VectorSubcoreMesh(
        core_axis_name="c", subcore_axis_name="s",
        num_cores=_NC, num_subcores=_NS),
    scratch_types=[
        pltpu.VMEM((_EW,), jnp.int32),
        pltpu.VMEM((_EW,), jnp.int32),
        pltpu.VMEM((_EW,), jnp.float32),
        pltpu.VMEM((_NCH, _CH), jnp.int32),
        pltpu.VMEM((_EW,), jnp.int32),
        pltpu.VMEM((_EW,), jnp.float32),
        pltpu.VMEM((_NCH, _CH), jnp.float32),
        pltpu.VMEM((_L,), jnp.float32),
        pltpu.VMEM((_ZCH,), jnp.float32),
        pltpu.VMEM_SHARED((_N,), jnp.float32),
        pltpu.SemaphoreType.DMA,
        pltpu.SemaphoreType.DMA,
        pltpu.SemaphoreType.DMA,
        pltpu.SemaphoreType.DMA,
    ],
)(_sc_body)


def _tc_reduce(h_ref, pd_ref, out_ref):
    wd = h_ref[0:_N] + h_ref[_N:2 * _N]
    d = wd - 2.0
    loss_deg = jnp.sum(d * d) * (1.0 / _N)
    loss_dist = jnp.sum(pd_ref[...])
    out_ref[0, 0] = loss_dist + _LAMBDA * loss_deg


_tc_call = pl.pallas_call(
    _tc_reduce,
    out_shape=jax.ShapeDtypeStruct((1, 1), jnp.float32),
    out_specs=pl.BlockSpec(memory_space=pltpu.SMEM),
)


def kernel(edge_logits, distances, edge_index, num_nodes):
    del num_nodes
    src = edge_index[0].astype(jnp.int32)
    dst = edge_index[1].astype(jnp.int32)
    dist_flat = (distances.reshape(_N // 8, 8, _N // 128, 128)
                 .transpose(0, 2, 1, 3).reshape(_N * _N))
    hist, pd = _sc_call(dist_flat, src, dst, edge_logits)
    res = _tc_call(hist, pd)
    return res[0, 0]
